# Initial kernel scaffold; baseline (speedup 1.0000x reference)
import jax
import jax.numpy as jnp
from jax.experimental import pallas as pl


def kernel(new_feats, edge_index, edge_weight, W1, W2):
    raise NotImplementedError("write your pallas kernel here")


# speedup vs baseline: 6.4295x; 6.4295x over previous
import functools

import jax
import jax.numpy as jnp
from jax import lax
from jax.experimental import pallas as pl
from jax.experimental.pallas import tpu as pltpu
from jax.experimental.pallas import tpu_sc as plsc

N = 10000
E = 160000
D = 256
DH = D // 2
NC = 2
NS = 16
EPT = E // NS
K = 80
CHUNKS = EPT // K
BLKS = 5
CPB = CHUNKS // BLKS
NPAD = 10112
RPT = NPAD // NS
ZR = 80


_BM = 1000


def _tc1_body(x_ref, w_ref, h_ref):
    h = jnp.dot(x_ref[...], w_ref[...], preferred_element_type=jnp.float32)
    h_ref[0] = h[:, :DH]
    h_ref[1] = h[:, DH:]


def _tc1(x, w):
    return pl.pallas_call(
        _tc1_body,
        grid=(N // _BM,),
        in_specs=[
            pl.BlockSpec((_BM, D), lambda i: (i, 0)),
            pl.BlockSpec((D, D), lambda i: (0, 0)),
        ],
        out_specs=pl.BlockSpec((NC, _BM, DH), lambda i: (0, i, 0)),
        out_shape=jax.ShapeDtypeStruct((NC, N, DH), jnp.float32),
    )(x, w)


def _tc2_body(nf_ref, out1_ref, b2_ref, w_ref, fin_ref):
    fin_ref[:, :D] = nf_ref[...]
    fin_ref[:, D:2 * D] = jnp.concatenate(
        [out1_ref[0], out1_ref[1]], axis=-1)
    b2 = jnp.concatenate([b2_ref[0], b2_ref[1]], axis=-1)
    h2 = jnp.dot(b2, w_ref[...], preferred_element_type=jnp.float32)
    fin_ref[:, 2 * D:] = jnp.tanh(h2)


def _tc2(nf, out1, b2, w):
    return pl.pallas_call(
        _tc2_body,
        grid=(N // _BM,),
        in_specs=[
            pl.BlockSpec((_BM, D), lambda i: (i, 0)),
            pl.BlockSpec((NC, _BM, DH), lambda i: (0, i, 0)),
            pl.BlockSpec((NC, _BM, DH), lambda i: (0, i, 0)),
            pl.BlockSpec((D, D), lambda i: (0, 0)),
        ],
        out_specs=pl.BlockSpec((_BM, 3 * D), lambda i: (i, 0)),
        out_shape=jax.ShapeDtypeStruct((N, 3 * D), jnp.float32),
    )(nf, out1, b2, w)


def _sc_body(h_ref, src_ref, dst_ref, w_ref, out1_ref, b2_ref,
             src_v, dst_v, w_v, rows_a, rows_b, agg_sp, sem_a, sem_b):
    c = lax.axis_index("c")
    s = lax.axis_index("s")

    def _zero_share():
        def _zrow(r, _):
            z = jnp.zeros((16,), jnp.float32)
            for j in range(DH // 16):
                rows_a[r, pl.ds(j * 16, 16)] = z
            return 0
        lax.fori_loop(0, ZR, _zrow, 0)
        for k in range(RPT // ZR):
            pltpu.sync_copy(rows_a.at[pl.ds(0, ZR)],
                            agg_sp.at[pl.ds(s * RPT + k * ZR, ZR)])
        rem = RPT % ZR
        if rem:
            pltpu.sync_copy(rows_a.at[pl.ds(0, rem)],
                            agg_sp.at[pl.ds(s * RPT + (RPT // ZR) * ZR, rem)])

    def _segment_sum(table_ref):
        def _gather(i, rows, sem):
            pltpu.async_copy(table_ref.at[c].at[src_v.at[i]], rows, sem)

        def _consume(i, rows, sem):
            pltpu.make_async_copy(
                table_ref.at[c].at[src_v.at[i]], rows, sem).wait()

            def _grp(g, _):
                wv = w_v[i, pl.ds(g * 16, 16)]
                for e in range(16):
                    w = jnp.full((16,), wv[e], jnp.float32)
                    row = g * 16 + e
                    for j in range(DH // 16):
                        sl = pl.ds(j * 16, 16)
                        rows[row, sl] = rows[row, sl] * w
                return 0
            lax.fori_loop(0, K // 16, _grp, 0)

            pltpu.sync_copy(rows, agg_sp.at[dst_v.at[i]], add=True)

        assert CPB % 2 == 1

        def _pair(p, _):
            i0 = 2 * p
            i1 = i0 + 1
            _gather(i1, rows_b, sem_b)
            _consume(i0, rows_a, sem_a)
            _gather(i1 + 1, rows_a, sem_a)
            _consume(i1, rows_b, sem_b)
            return 0

        def _blk(bk, _):
            pltpu.sync_copy(src_ref.at[s, bk], src_v)
            pltpu.sync_copy(dst_ref.at[s, bk], dst_v)
            pltpu.sync_copy(w_ref.at[s, bk], w_v)
            _gather(0, rows_a, sem_a)
            lax.fori_loop(0, CPB // 2, _pair, 0)
            _consume(CPB - 1, rows_a, sem_a)
            return 0

        lax.fori_loop(0, BLKS, _blk, 0)

    _zero_share()
    plsc.subcore_barrier()
    _segment_sum(h_ref)
    plsc.subcore_barrier()

    def _tanh_rows(nrows, sp_off):
        pltpu.sync_copy(agg_sp.at[pl.ds(sp_off, nrows)],
                        rows_b.at[pl.ds(0, nrows)])

        def _trow(r, _):
            for j in range(DH // 16):
                sl = pl.ds(j * 16, 16)
                x = rows_b[r, sl]
                e = jnp.exp(jnp.abs(x) * -2.0)
                t = (1.0 - e) / (1.0 + e)
                rows_b[r, sl] = jnp.where(x < 0.0, -t, t)
            return 0
        lax.fori_loop(0, nrows, _trow, 0)
        pltpu.sync_copy(rows_b.at[pl.ds(0, nrows)],
                        out1_ref.at[c, pl.ds(sp_off, nrows)])

    for k in range(RPT // ZR):
        _tanh_rows(ZR, s * RPT + k * ZR)
    if RPT % ZR:
        _tanh_rows(RPT % ZR, s * RPT + (RPT // ZR) * ZR)

    _zero_share()
    plsc.subcore_barrier()
    _segment_sum(out1_ref)
    plsc.subcore_barrier()

    pltpu.sync_copy(agg_sp.at[pl.ds(s * RPT, RPT)],
                    b2_ref.at[c, pl.ds(s * RPT, RPT)])


_sc_both_layers = pl.kernel(
    _sc_body,
    out_type=(
        jax.ShapeDtypeStruct((NC, NPAD, DH), jnp.float32),
        jax.ShapeDtypeStruct((NC, NPAD, DH), jnp.float32),
    ),
    mesh=plsc.VectorSubcoreMesh(core_axis_name="c", subcore_axis_name="s"),
    scratch_types=[
        pltpu.VMEM((CPB, K), jnp.int32),
        pltpu.VMEM((CPB, K), jnp.int32),
        pltpu.VMEM((CPB, K), jnp.float32),
        pltpu.VMEM((K, DH), jnp.float32),
        pltpu.VMEM((K, DH), jnp.float32),
        pltpu.VMEM_SHARED((NPAD, DH), jnp.float32),
        pltpu.SemaphoreType.DMA,
        pltpu.SemaphoreType.DMA,
    ],
    name="sc_gcn_both_layers",
)


@jax.jit
def kernel(new_feats, edge_index, edge_weight, W1, W2):
    src = edge_index[1].reshape(NS, BLKS, CPB, K)
    dst = edge_index[0].reshape(NS, BLKS, CPB, K)
    w = edge_weight.reshape(NS, BLKS, CPB, K)

    h1 = _tc1(new_feats, W1)
    out1, b2 = _sc_both_layers(h1, src, dst, w)
    return _tc2(new_feats, out1, b2, W2)
